# Initial kernel scaffold; baseline (speedup 1.0000x reference)
import jax
import jax.numpy as jnp
from jax.experimental import pallas as pl


def kernel(x, adj, W1, W2, W3, fea_weight, bn_gamma, bn_beta):
    raise NotImplementedError("write your pallas kernel here")


# speedup vs baseline: 1.2428x; 1.2428x over previous
import functools

import jax
import jax.numpy as jnp
from jax.experimental import pallas as pl


def _bn_gemm_kernel(x_ref, w1_ref, gamma_ref, beta_ref, y1_ref):
    x = x_ref[...]
    mean = jnp.mean(x, axis=0, keepdims=True)
    var = jnp.mean((x - mean) ** 2, axis=0, keepdims=True)
    scale = gamma_ref[...] / jnp.sqrt(var + 1e-5)
    xh = (x - mean) * scale + beta_ref[...]
    y1_ref[...] = jnp.dot(xh, w1_ref[...], preferred_element_type=jnp.float32)


def _gc1_kernel(adj_ref, y1_ref, w23_ref, y2_ref):
    h = jnp.dot(adj_ref[...], y1_ref[...], preferred_element_type=jnp.float32)
    h = jnp.maximum(h, 0.0)
    y2_ref[...] = jnp.dot(h, w23_ref[...], preferred_element_type=jnp.float32)


def _gc23_kernel(adj_ref, y2_ref, few_ref, mu_ref, lv_ref, xp_ref, *, h2):
    mv = jnp.dot(adj_ref[...], y2_ref[...], preferred_element_type=jnp.float32)
    mu = mv[:, :h2]
    mu_ref[...] = mu
    lv_ref[...] = mv[:, h2:]
    xp = jnp.dot(mu, few_ref[...], preferred_element_type=jnp.float32)
    xp_ref[...] = jnp.where(xp >= 0, xp, 0.01 * xp)


def _transpose_kernel(mu_ref, mut_ref):
    mut_ref[...] = mu_ref[...].T


def _apred_kernel(mu_ref, mut_ref, a_ref):
    a_ref[...] = jnp.dot(mu_ref[...], mut_ref[...],
                         preferred_element_type=jnp.float32)


def kernel(x, adj, W1, W2, W3, fea_weight, bn_gamma, bn_beta):
    n, d = x.shape
    h1 = W1.shape[1]
    h2 = W2.shape[1]
    f32 = jnp.float32

    bm = 400 if n % 400 == 0 else n
    grid = n // bm

    w23 = jnp.concatenate([W2, W3], axis=1)
    gamma2 = bn_gamma.reshape(1, d)
    beta2 = bn_beta.reshape(1, d)

    y1 = pl.pallas_call(
        _bn_gemm_kernel,
        out_shape=jax.ShapeDtypeStruct((n, h1), f32),
    )(x, W1, gamma2, beta2)

    y2 = pl.pallas_call(
        _gc1_kernel,
        grid=(grid,),
        in_specs=[
            pl.BlockSpec((bm, n), lambda i: (i, 0)),
            pl.BlockSpec((n, h1), lambda i: (0, 0)),
            pl.BlockSpec((h1, 2 * h2), lambda i: (0, 0)),
        ],
        out_specs=pl.BlockSpec((bm, 2 * h2), lambda i: (i, 0)),
        out_shape=jax.ShapeDtypeStruct((n, 2 * h2), f32),
    )(adj, y1, w23)

    mu, logvar, x_pred = pl.pallas_call(
        functools.partial(_gc23_kernel, h2=h2),
        grid=(grid,),
        in_specs=[
            pl.BlockSpec((bm, n), lambda i: (i, 0)),
            pl.BlockSpec((n, 2 * h2), lambda i: (0, 0)),
            pl.BlockSpec((h2, d), lambda i: (0, 0)),
        ],
        out_specs=[
            pl.BlockSpec((bm, h2), lambda i: (i, 0)),
            pl.BlockSpec((bm, h2), lambda i: (i, 0)),
            pl.BlockSpec((bm, d), lambda i: (i, 0)),
        ],
        out_shape=[
            jax.ShapeDtypeStruct((n, h2), f32),
            jax.ShapeDtypeStruct((n, h2), f32),
            jax.ShapeDtypeStruct((n, d), f32),
        ],
    )(adj, y2, fea_weight)

    mu_t = pl.pallas_call(
        _transpose_kernel,
        out_shape=jax.ShapeDtypeStruct((h2, n), f32),
    )(mu)

    a_pred = pl.pallas_call(
        _apred_kernel,
        grid=(grid,),
        in_specs=[
            pl.BlockSpec((bm, h2), lambda i: (i, 0)),
            pl.BlockSpec((h2, n), lambda i: (0, 0)),
        ],
        out_specs=pl.BlockSpec((bm, n), lambda i: (i, 0)),
        out_shape=jax.ShapeDtypeStruct((n, n), f32),
    )(mu, mu_t)

    return (a_pred, x_pred, mu, logvar, mu)


# speedup vs baseline: 1.2691x; 1.0212x over previous
import functools

import jax
import jax.numpy as jnp
from jax.experimental import pallas as pl
from jax.experimental.pallas import tpu as pltpu


def _gc1_kernel(x_ref, w1_ref, gamma_ref, beta_ref, adj_ref, w23_ref,
                y2_ref, y1_scr):
    @pl.when(pl.program_id(0) == 0)
    def _prologue():
        x = x_ref[...]
        mean = jnp.mean(x, axis=0, keepdims=True)
        var = jnp.mean((x - mean) ** 2, axis=0, keepdims=True)
        scale = gamma_ref[...] / jnp.sqrt(var + 1e-5)
        xh = (x - mean) * scale + beta_ref[...]
        y1_scr[...] = jnp.dot(xh, w1_ref[...],
                              preferred_element_type=jnp.float32)

    h = jnp.dot(adj_ref[...], y1_scr[...],
                preferred_element_type=jnp.float32)
    h = jnp.maximum(h, 0.0)
    y2_ref[...] = jnp.dot(h, w23_ref[...],
                          preferred_element_type=jnp.float32)


def _gc23_kernel(adj_ref, y2_ref, few_ref, mu_ref, lv_ref, xp_ref, *, h2):
    mv = jnp.dot(adj_ref[...], y2_ref[...],
                 preferred_element_type=jnp.float32)
    mu = mv[:, :h2]
    mu_ref[...] = mu
    lv_ref[...] = mv[:, h2:]
    xp = jnp.dot(mu, few_ref[...], preferred_element_type=jnp.float32)
    xp_ref[...] = jnp.where(xp >= 0, xp, 0.01 * xp)


def _apred_kernel(mu_full_ref, mu_ref, a_ref, mut_scr):
    @pl.when(pl.program_id(0) == 0)
    def _prologue():
        mut_scr[...] = mu_full_ref[...].T

    a_ref[...] = jnp.dot(mu_ref[...], mut_scr[...],
                         preferred_element_type=jnp.float32)


def kernel(x, adj, W1, W2, W3, fea_weight, bn_gamma, bn_beta):
    n, d = x.shape
    h1 = W1.shape[1]
    h2 = W2.shape[1]
    f32 = jnp.float32

    bm = 400 if n % 400 == 0 else n
    grid = n // bm

    w23 = jnp.concatenate([W2, W3], axis=1)
    gamma2 = bn_gamma.reshape(1, d)
    beta2 = bn_beta.reshape(1, d)

    y2 = pl.pallas_call(
        _gc1_kernel,
        grid=(grid,),
        in_specs=[
            pl.BlockSpec((n, d), lambda i: (0, 0)),
            pl.BlockSpec((d, h1), lambda i: (0, 0)),
            pl.BlockSpec((1, d), lambda i: (0, 0)),
            pl.BlockSpec((1, d), lambda i: (0, 0)),
            pl.BlockSpec((bm, n), lambda i: (i, 0)),
            pl.BlockSpec((h1, 2 * h2), lambda i: (0, 0)),
        ],
        out_specs=pl.BlockSpec((bm, 2 * h2), lambda i: (i, 0)),
        out_shape=jax.ShapeDtypeStruct((n, 2 * h2), f32),
        scratch_shapes=[pltpu.VMEM((n, h1), f32)],
    )(x, W1, gamma2, beta2, adj, w23)

    mu, logvar, x_pred = pl.pallas_call(
        functools.partial(_gc23_kernel, h2=h2),
        grid=(grid,),
        in_specs=[
            pl.BlockSpec((bm, n), lambda i: (i, 0)),
            pl.BlockSpec((n, 2 * h2), lambda i: (0, 0)),
            pl.BlockSpec((h2, d), lambda i: (0, 0)),
        ],
        out_specs=[
            pl.BlockSpec((bm, h2), lambda i: (i, 0)),
            pl.BlockSpec((bm, h2), lambda i: (i, 0)),
            pl.BlockSpec((bm, d), lambda i: (i, 0)),
        ],
        out_shape=[
            jax.ShapeDtypeStruct((n, h2), f32),
            jax.ShapeDtypeStruct((n, h2), f32),
            jax.ShapeDtypeStruct((n, d), f32),
        ],
    )(adj, y2, fea_weight)

    a_pred = pl.pallas_call(
        _apred_kernel,
        grid=(grid,),
        in_specs=[
            pl.BlockSpec((n, h2), lambda i: (0, 0)),
            pl.BlockSpec((bm, h2), lambda i: (i, 0)),
        ],
        out_specs=pl.BlockSpec((bm, n), lambda i: (i, 0)),
        out_shape=jax.ShapeDtypeStruct((n, n), f32),
        scratch_shapes=[pltpu.VMEM((h2, n), f32)],
    )(mu, mu)

    return (a_pred, x_pred, mu, logvar, mu)


# speedup vs baseline: 1.2811x; 1.0095x over previous
import functools

import jax
import jax.numpy as jnp
from jax.experimental import pallas as pl
from jax.experimental.pallas import tpu as pltpu


def _mega_kernel(x_ref, w1_ref, gamma_ref, beta_ref, adj_ref, w23_ref,
                 few_ref, mu_ref, lv_ref, xp_ref, a_ref,
                 y1_scr, y2_scr, mu_scr, mut_scr, *, g, bm, h2):
    i = pl.program_id(0)

    @pl.when(i == 0)
    def _bn_prologue():
        x = x_ref[...]
        mean = jnp.mean(x, axis=0, keepdims=True)
        var = jnp.mean((x - mean) ** 2, axis=0, keepdims=True)
        scale = gamma_ref[...] / jnp.sqrt(var + 1e-5)
        xh = (x - mean) * scale + beta_ref[...]
        y1_scr[...] = jnp.dot(xh, w1_ref[...],
                              preferred_element_type=jnp.float32)

    @pl.when(i < g)
    def _phase0():
        r = i
        h = jnp.dot(adj_ref[...], y1_scr[...],
                    preferred_element_type=jnp.float32)
        h = jnp.maximum(h, 0.0)
        y2_scr[pl.ds(r * bm, bm), :] = jnp.dot(
            h, w23_ref[...], preferred_element_type=jnp.float32)

    @pl.when((i >= g) & (i < 2 * g))
    def _phase1():
        r = i - g
        mv = jnp.dot(adj_ref[...], y2_scr[...],
                     preferred_element_type=jnp.float32)
        mu = mv[:, :h2]
        mu_ref[...] = mu
        lv_ref[...] = mv[:, h2:]
        mu_scr[pl.ds(r * bm, bm), :] = mu
        xp = jnp.dot(mu, few_ref[...], preferred_element_type=jnp.float32)
        xp_ref[...] = jnp.where(xp >= 0, xp, 0.01 * xp)

    @pl.when(i >= 2 * g)
    def _phase2():
        r = i - 2 * g

        @pl.when(r == 0)
        def _transpose():
            mut_scr[...] = mu_scr[...].T

        a_ref[...] = jnp.dot(mu_scr[pl.ds(r * bm, bm), :], mut_scr[...],
                             preferred_element_type=jnp.float32)


def kernel(x, adj, W1, W2, W3, fea_weight, bn_gamma, bn_beta):
    n, d = x.shape
    h1 = W1.shape[1]
    h2 = W2.shape[1]
    f32 = jnp.float32

    bm = 200 if n % 200 == 0 else n
    g = n // bm

    w23 = jnp.concatenate([W2, W3], axis=1)
    gamma2 = bn_gamma.reshape(1, d)
    beta2 = bn_beta.reshape(1, d)

    def adj_map(i):
        return (jnp.where(i < g, i,
                          jnp.where(i < 2 * g, i - g, g - 1)), 0)

    def p1_map(i):
        return (jnp.where(i < g, 0,
                          jnp.where(i < 2 * g, i - g, g - 1)), 0)

    def p2_map(i):
        return (jnp.where(i < 2 * g, 0, i - 2 * g), 0)

    mu, logvar, x_pred, a_pred = pl.pallas_call(
        functools.partial(_mega_kernel, g=g, bm=bm, h2=h2),
        grid=(3 * g,),
        in_specs=[
            pl.BlockSpec((n, d), lambda i: (0, 0)),
            pl.BlockSpec((d, h1), lambda i: (0, 0)),
            pl.BlockSpec((1, d), lambda i: (0, 0)),
            pl.BlockSpec((1, d), lambda i: (0, 0)),
            pl.BlockSpec((bm, n), adj_map),
            pl.BlockSpec((h1, 2 * h2), lambda i: (0, 0)),
            pl.BlockSpec((h2, d), lambda i: (0, 0)),
        ],
        out_specs=[
            pl.BlockSpec((bm, h2), p1_map),
            pl.BlockSpec((bm, h2), p1_map),
            pl.BlockSpec((bm, d), p1_map),
            pl.BlockSpec((bm, n), p2_map),
        ],
        out_shape=[
            jax.ShapeDtypeStruct((n, h2), f32),
            jax.ShapeDtypeStruct((n, h2), f32),
            jax.ShapeDtypeStruct((n, d), f32),
            jax.ShapeDtypeStruct((n, n), f32),
        ],
        scratch_shapes=[
            pltpu.VMEM((n, h1), f32),
            pltpu.VMEM((n, 2 * h2), f32),
            pltpu.VMEM((n, h2), f32),
            pltpu.VMEM((h2, n), f32),
        ],
    )(x, W1, gamma2, beta2, adj, w23, fea_weight)

    return (a_pred, x_pred, mu, logvar, mu)


# speedup vs baseline: 1.2970x; 1.0124x over previous
import functools

import jax
import jax.numpy as jnp
from jax.experimental import pallas as pl
from jax.experimental.pallas import tpu as pltpu


def _enc_kernel(x_ref, w1_ref, gamma_ref, beta_ref, adj_ref, w23_ref,
                few_ref, mu_ref, lv_ref, xp_ref,
                y1_scr, y2_scr, *, g, bm, h2):
    i = pl.program_id(0)

    @pl.when(i == 0)
    def _bn_prologue():
        x = x_ref[...]
        mean = jnp.mean(x, axis=0, keepdims=True)
        var = jnp.mean((x - mean) ** 2, axis=0, keepdims=True)
        scale = gamma_ref[...] / jnp.sqrt(var + 1e-5)
        xh = (x - mean) * scale + beta_ref[...]
        y1_scr[...] = jnp.dot(xh, w1_ref[...],
                              preferred_element_type=jnp.float32)

    @pl.when(i < g)
    def _phase0():
        r = i
        h = jnp.dot(adj_ref[...], y1_scr[...],
                    preferred_element_type=jnp.float32)
        h = jnp.maximum(h, 0.0)
        y2_scr[pl.ds(r * bm, bm), :] = jnp.dot(
            h, w23_ref[...], preferred_element_type=jnp.float32)

    @pl.when(i >= g)
    def _phase1():
        mv = jnp.dot(adj_ref[...], y2_scr[...],
                     preferred_element_type=jnp.float32)
        mu = mv[:, :h2]
        mu_ref[...] = mu
        lv_ref[...] = mv[:, h2:]
        xp = jnp.dot(mu, few_ref[...], preferred_element_type=jnp.float32)
        xp_ref[...] = jnp.where(xp >= 0, xp, 0.01 * xp)


def _apred_kernel(mu_full_ref, mu_ref, a_ref, mut_scr):
    @pl.when(pl.program_id(0) == 0)
    def _prologue():
        mut_scr[...] = mu_full_ref[...].T

    a_ref[...] = jnp.dot(mu_ref[...], mut_scr[...],
                         preferred_element_type=jnp.float32)


def kernel(x, adj, W1, W2, W3, fea_weight, bn_gamma, bn_beta):
    n, d = x.shape
    h1 = W1.shape[1]
    h2 = W2.shape[1]
    f32 = jnp.float32

    bm = 400 if n % 400 == 0 else n
    g = n // bm

    w23 = jnp.concatenate([W2, W3], axis=1)
    gamma2 = bn_gamma.reshape(1, d)
    beta2 = bn_beta.reshape(1, d)

    def adj_map(i):
        return (jnp.where(i < g, i, 2 * g - 1 - i), 0)

    def p1_map(i):
        return (jnp.where(i < g, g - 1, 2 * g - 1 - i), 0)

    mu, logvar, x_pred = pl.pallas_call(
        functools.partial(_enc_kernel, g=g, bm=bm, h2=h2),
        grid=(2 * g,),
        in_specs=[
            pl.BlockSpec((n, d), lambda i: (0, 0)),
            pl.BlockSpec((d, h1), lambda i: (0, 0)),
            pl.BlockSpec((1, d), lambda i: (0, 0)),
            pl.BlockSpec((1, d), lambda i: (0, 0)),
            pl.BlockSpec((bm, n), adj_map),
            pl.BlockSpec((h1, 2 * h2), lambda i: (0, 0)),
            pl.BlockSpec((h2, d), lambda i: (0, 0)),
        ],
        out_specs=[
            pl.BlockSpec((bm, h2), p1_map),
            pl.BlockSpec((bm, h2), p1_map),
            pl.BlockSpec((bm, d), p1_map),
        ],
        out_shape=[
            jax.ShapeDtypeStruct((n, h2), f32),
            jax.ShapeDtypeStruct((n, h2), f32),
            jax.ShapeDtypeStruct((n, d), f32),
        ],
        scratch_shapes=[
            pltpu.VMEM((n, h1), f32),
            pltpu.VMEM((n, 2 * h2), f32),
        ],
    )(x, W1, gamma2, beta2, adj, w23, fea_weight)

    a_pred = pl.pallas_call(
        _apred_kernel,
        grid=(g,),
        in_specs=[
            pl.BlockSpec((n, h2), lambda i: (0, 0)),
            pl.BlockSpec((bm, h2), lambda i: (i, 0)),
        ],
        out_specs=pl.BlockSpec((bm, n), lambda i: (i, 0)),
        out_shape=jax.ShapeDtypeStruct((n, n), f32),
        scratch_shapes=[pltpu.VMEM((h2, n), f32)],
    )(mu, mu)

    return (a_pred, x_pred, mu, logvar, mu)


# speedup vs baseline: 1.2975x; 1.0004x over previous
import functools

import jax
import jax.numpy as jnp
from jax.experimental import pallas as pl
from jax.experimental.pallas import tpu as pltpu


def _enc_kernel(x_ref, w1_ref, gamma_ref, beta_ref, adj_ref, w23_ref,
                few_ref, mu_ref, lv_ref, xp_ref,
                y1_scr, y2_scr, *, g, bm, h2):
    i = pl.program_id(0)

    @pl.when(i == 0)
    def _bn_prologue():
        x = x_ref[...]
        mean = jnp.mean(x, axis=0, keepdims=True)
        var = jnp.mean((x - mean) ** 2, axis=0, keepdims=True)
        scale = gamma_ref[...] / jnp.sqrt(var + 1e-5)
        xh = (x - mean) * scale + beta_ref[...]
        y1_scr[...] = jnp.dot(xh, w1_ref[...],
                              preferred_element_type=jnp.float32
                              ).astype(jnp.bfloat16)

    @pl.when(i < g)
    def _phase0():
        r = i
        h = jnp.dot(adj_ref[...].astype(jnp.bfloat16), y1_scr[...],
                    preferred_element_type=jnp.float32)
        h = jnp.maximum(h, 0.0)
        y2_scr[pl.ds(r * bm, bm), :] = jnp.dot(
            h, w23_ref[...], preferred_element_type=jnp.float32
        ).astype(jnp.bfloat16)

    @pl.when(i >= g)
    def _phase1():
        mv = jnp.dot(adj_ref[...].astype(jnp.bfloat16), y2_scr[...],
                     preferred_element_type=jnp.float32)
        mu = mv[:, :h2]
        mu_ref[...] = mu
        lv_ref[...] = mv[:, h2:]
        xp = jnp.dot(mu, few_ref[...], preferred_element_type=jnp.float32)
        xp_ref[...] = jnp.where(xp >= 0, xp, 0.01 * xp)


def _apred_kernel(mu_full_ref, mu_ref, a_ref, mut_scr):
    @pl.when(pl.program_id(0) == 0)
    def _prologue():
        mut_scr[...] = mu_full_ref[...].T

    a_ref[...] = jnp.dot(mu_ref[...], mut_scr[...],
                         preferred_element_type=jnp.float32)


def kernel(x, adj, W1, W2, W3, fea_weight, bn_gamma, bn_beta):
    n, d = x.shape
    h1 = W1.shape[1]
    h2 = W2.shape[1]
    f32 = jnp.float32

    bm = 400 if n % 400 == 0 else n
    g = n // bm

    w23 = jnp.concatenate([W2, W3], axis=1)
    gamma2 = bn_gamma.reshape(1, d)
    beta2 = bn_beta.reshape(1, d)

    def adj_map(i):
        return (jnp.where(i < g, i, 2 * g - 1 - i), 0)

    def p1_map(i):
        return (jnp.where(i < g, g - 1, 2 * g - 1 - i), 0)

    mu, logvar, x_pred = pl.pallas_call(
        functools.partial(_enc_kernel, g=g, bm=bm, h2=h2),
        grid=(2 * g,),
        in_specs=[
            pl.BlockSpec((n, d), lambda i: (0, 0)),
            pl.BlockSpec((d, h1), lambda i: (0, 0)),
            pl.BlockSpec((1, d), lambda i: (0, 0)),
            pl.BlockSpec((1, d), lambda i: (0, 0)),
            pl.BlockSpec((bm, n), adj_map),
            pl.BlockSpec((h1, 2 * h2), lambda i: (0, 0)),
            pl.BlockSpec((h2, d), lambda i: (0, 0)),
        ],
        out_specs=[
            pl.BlockSpec((bm, h2), p1_map),
            pl.BlockSpec((bm, h2), p1_map),
            pl.BlockSpec((bm, d), p1_map),
        ],
        out_shape=[
            jax.ShapeDtypeStruct((n, h2), f32),
            jax.ShapeDtypeStruct((n, h2), f32),
            jax.ShapeDtypeStruct((n, d), f32),
        ],
        scratch_shapes=[
            pltpu.VMEM((n, h1), jnp.bfloat16),
            pltpu.VMEM((n, 2 * h2), jnp.bfloat16),
        ],
    )(x, W1, gamma2, beta2, adj, w23, fea_weight)

    a_pred = pl.pallas_call(
        _apred_kernel,
        grid=(g,),
        in_specs=[
            pl.BlockSpec((n, h2), lambda i: (0, 0)),
            pl.BlockSpec((bm, h2), lambda i: (i, 0)),
        ],
        out_specs=pl.BlockSpec((bm, n), lambda i: (i, 0)),
        out_shape=jax.ShapeDtypeStruct((n, n), f32),
        scratch_shapes=[pltpu.VMEM((h2, n), f32)],
    )(mu, mu)

    return (a_pred, x_pred, mu, logvar, mu)


# speedup vs baseline: 1.3017x; 1.0033x over previous
import functools

import jax
import jax.numpy as jnp
from jax.experimental import pallas as pl
from jax.experimental.pallas import tpu as pltpu


def _enc_kernel(x_ref, w1_ref, gamma_ref, beta_ref, adj_ref, w23_ref,
                few_ref, mu_ref, lv_ref, xp_ref,
                y1_scr, y2_scr, *, g, bm, h2):
    i = pl.program_id(0)

    @pl.when(i == 0)
    def _bn_prologue():
        x = x_ref[...]
        mean = jnp.mean(x, axis=0, keepdims=True)
        var = jnp.mean((x - mean) ** 2, axis=0, keepdims=True)
        scale = gamma_ref[...] / jnp.sqrt(var + 1e-5)
        xh = (x - mean) * scale + beta_ref[...]
        y1_scr[...] = jnp.dot(xh, w1_ref[...],
                              preferred_element_type=jnp.float32
                              ).astype(jnp.bfloat16)

    @pl.when(i < g)
    def _phase0():
        r = i
        h = jnp.dot(adj_ref[...].astype(jnp.bfloat16), y1_scr[...],
                    preferred_element_type=jnp.float32)
        h = jnp.maximum(h, 0.0)
        y2_scr[pl.ds(r * bm, bm), :] = jnp.dot(
            h, w23_ref[...], preferred_element_type=jnp.float32
        ).astype(jnp.bfloat16)

    @pl.when(i >= g)
    def _phase1():
        mv = jnp.dot(adj_ref[...].astype(jnp.bfloat16), y2_scr[...],
                     preferred_element_type=jnp.float32)
        mu = mv[:, :h2]
        mu_ref[...] = mu
        lv_ref[...] = mv[:, h2:]
        xp = jnp.dot(mu, few_ref[...], preferred_element_type=jnp.float32)
        xp_ref[...] = jnp.where(xp >= 0, xp, 0.01 * xp)


def _apred_kernel(mu_full_ref, mu_ref, a_ref):
    a_ref[...] = jax.lax.dot_general(
        mu_ref[...], mu_full_ref[...],
        dimension_numbers=(((1,), (1,)), ((), ())),
        preferred_element_type=jnp.float32)


def kernel(x, adj, W1, W2, W3, fea_weight, bn_gamma, bn_beta):
    n, d = x.shape
    h1 = W1.shape[1]
    h2 = W2.shape[1]
    f32 = jnp.float32

    bm = 400 if n % 400 == 0 else n
    g = n // bm

    w23 = jnp.concatenate([W2, W3], axis=1)
    gamma2 = bn_gamma.reshape(1, d)
    beta2 = bn_beta.reshape(1, d)

    def adj_map(i):
        return (jnp.where(i < g, i, 2 * g - 1 - i), 0)

    def p1_map(i):
        return (jnp.where(i < g, g - 1, 2 * g - 1 - i), 0)

    mu, logvar, x_pred = pl.pallas_call(
        functools.partial(_enc_kernel, g=g, bm=bm, h2=h2),
        grid=(2 * g,),
        in_specs=[
            pl.BlockSpec((n, d), lambda i: (0, 0)),
            pl.BlockSpec((d, h1), lambda i: (0, 0)),
            pl.BlockSpec((1, d), lambda i: (0, 0)),
            pl.BlockSpec((1, d), lambda i: (0, 0)),
            pl.BlockSpec((bm, n), adj_map),
            pl.BlockSpec((h1, 2 * h2), lambda i: (0, 0)),
            pl.BlockSpec((h2, d), lambda i: (0, 0)),
        ],
        out_specs=[
            pl.BlockSpec((bm, h2), p1_map),
            pl.BlockSpec((bm, h2), p1_map),
            pl.BlockSpec((bm, d), p1_map),
        ],
        out_shape=[
            jax.ShapeDtypeStruct((n, h2), f32),
            jax.ShapeDtypeStruct((n, h2), f32),
            jax.ShapeDtypeStruct((n, d), f32),
        ],
        scratch_shapes=[
            pltpu.VMEM((n, h1), jnp.bfloat16),
            pltpu.VMEM((n, 2 * h2), jnp.bfloat16),
        ],
    )(x, W1, gamma2, beta2, adj, w23, fea_weight)

    a_pred = pl.pallas_call(
        _apred_kernel,
        grid=(g,),
        in_specs=[
            pl.BlockSpec((n, h2), lambda i: (0, 0)),
            pl.BlockSpec((bm, h2), lambda i: (i, 0)),
        ],
        out_specs=pl.BlockSpec((bm, n), lambda i: (i, 0)),
        out_shape=jax.ShapeDtypeStruct((n, n), f32),
    )(mu, mu)

    return (a_pred, x_pred, mu, logvar, mu)
